# baseline (speedup 1.0000x reference)
import functools

import jax
import jax.numpy as jnp
from jax import lax
from jax.experimental import pallas as pl
from jax.experimental.pallas import tpu as pltpu
from jax.experimental.pallas import tpu_sc as plsc

D_MODEL = 64
SCALE = 8.0

BATCH = 4096
SEQ = 200
NUM_CORES = 2
NUM_SUBCORES = 16
NW = NUM_CORES * NUM_SUBCORES
B_LOCAL = BATCH // NW
NBUF = 4
G1 = 128
G2 = SEQ - G1
ROW_UNROLL = 8


def _build():
  mesh = plsc.VectorSubcoreMesh(core_axis_name="c", subcore_axis_name="s")

  @functools.partial(
      pl.kernel,
      mesh=mesh,
      out_type=jax.ShapeDtypeStruct((BATCH, SEQ, D_MODEL), jnp.float32),
      scratch_types=[
          pltpu.VMEM((SEQ, B_LOCAL), jnp.int32),
          pltpu.VMEM((B_LOCAL * SEQ,), jnp.int32),
          pltpu.VMEM((NBUF, SEQ, D_MODEL), jnp.float32),
          [pltpu.SemaphoreType.DMA] * NBUF,
          [pltpu.SemaphoreType.DMA] * NBUF,
      ],
      compiler_params=pltpu.CompilerParams(
          use_tc_tiling_on_sc=False, needs_layout_passes=False
      ),
  )
  def emb(xt_hbm, table_hbm, out_hbm, xt_v, idx_v, rows_v, gsems, ssems):
    wid = lax.axis_index("s") * NUM_CORES + lax.axis_index("c")
    b0 = wid * B_LOCAL

    pltpu.sync_copy(xt_hbm.at[:, pl.ds(b0, B_LOCAL)], xt_v)

    lanes = lax.iota(jnp.int32, 16)

    def tr_body(l, carry):
      for k in range(13):
        j0 = k * 16 if k < 12 else SEQ - 16
        vec = plsc.load_gather(xt_v, [j0 + lanes, jnp.full((16,), l, jnp.int32)])
        idx_v[pl.ds(l * SEQ + j0, 16)] = vec
      return carry

    lax.fori_loop(0, B_LOCAL, tr_body, 0)

    def start_gather(b, l):
      pltpu.make_async_copy(
          table_hbm.at[idx_v.at[pl.ds(l * SEQ, G1)]],
          rows_v.at[b, pl.ds(0, G1)],
          gsems[b],
      ).start()
      pltpu.make_async_copy(
          table_hbm.at[idx_v.at[pl.ds(l * SEQ + G1, G2)]],
          rows_v.at[b, pl.ds(G1, G2)],
          gsems[b],
      ).start()

    def wait_gather(b, l):
      pltpu.make_async_copy(
          table_hbm.at[idx_v.at[pl.ds(l * SEQ, G1)]],
          rows_v.at[b, pl.ds(0, G1)],
          gsems[b],
      ).wait()
      pltpu.make_async_copy(
          table_hbm.at[idx_v.at[pl.ds(l * SEQ + G1, G2)]],
          rows_v.at[b, pl.ds(G1, G2)],
          gsems[b],
      ).wait()

    for b in range(NBUF):
      start_gather(b, b)

    def scale_buf(b):
      def body(i, c2):
        r0 = i * ROW_UNROLL
        for k in range(ROW_UNROLL):
          for j in range(D_MODEL // 16):
            s = pl.ds(j * 16, 16)
            rows_v[b, r0 + k, s] = rows_v[b, r0 + k, s] * SCALE
        return c2

      lax.fori_loop(0, SEQ // ROW_UNROLL, body, 0)

    def outer(i, carry):
      for b in range(NBUF):
        l = i * NBUF + b
        wait_gather(b, l)
        scale_buf(b)
        pltpu.make_async_copy(
            rows_v.at[b], out_hbm.at[b0 + l], ssems[b]
        ).start()

        @pl.when(l + NBUF < B_LOCAL)
        def _():
          pltpu.make_async_copy(
              rows_v.at[b], out_hbm.at[b0 + l], ssems[b]
          ).wait()
          start_gather(b, l + NBUF)

      return carry

    lax.fori_loop(0, B_LOCAL // NBUF, outer, 0)

    for b in range(NBUF):
      pltpu.make_async_copy(
          rows_v.at[b], out_hbm.at[b0], ssems[b]
      ).wait()

  return emb


_emb = _build()


@jax.jit
def kernel(x, lut):
  return _emb(x.T, lut)
